# baseline (speedup 1.0000x reference)
import functools

import jax
import jax.numpy as jnp
import numpy as np
from jax import lax
from jax.experimental import pallas as pl
from jax.experimental.pallas import tpu as pltpu
from jax.experimental.pallas import tpu_sc as plsc

NUM_USERS = 1000000
NUM_ITEMS = 1000000
EMB_DIM = 64
BATCH = 16384

_NC = 2
_NS = 16
_L = 16
_NW = _NC * _NS
_BPW = BATCH // _NW
_WPR = EMB_DIM // 2
_MASKHI = -65536


def _rsqrt16(s):
    s = jnp.maximum(s, jnp.float32(1e-24))
    i = lax.bitcast_convert_type(s, jnp.int32)
    i = jnp.int32(0x5F3759DF) - lax.shift_right_logical(i, 1)
    y = lax.bitcast_convert_type(i, jnp.float32)
    for _ in range(2):
        y = y * (jnp.float32(1.5) - jnp.float32(0.5) * s * y * y)
    return y


def _shuffle_xor(x, lanes, k):
    idx = lax.bitwise_xor(lanes, jnp.int32(k))
    return lax.gather(
        x, idx[:, None],
        dimension_numbers=lax.GatherDimensionNumbers(
            offset_dims=(), collapsed_slice_dims=(0,), start_index_map=(0,)),
        slice_sizes=(1,),
        mode=lax.GatherScatterMode.PROMISE_IN_BOUNDS)


def _unpack_pair(w):
    a = lax.bitcast_convert_type(lax.shift_left(w, 16), jnp.float32)
    b = lax.bitcast_convert_type(
        lax.bitwise_and(w, jnp.int32(_MASKHI)), jnp.float32)
    return a, b


@functools.cache
def _make_tower_kernel():
    mesh = plsc.VectorSubcoreMesh(core_axis_name="c", subcore_axis_name="s")

    @functools.partial(
        pl.kernel,
        mesh=mesh,
        out_type=jax.ShapeDtypeStruct((BATCH, EMB_DIM), jnp.float32),
        scratch_types=[
            pltpu.VMEM((_BPW,), jnp.int32),
            pltpu.VMEM((_BPW // 2, _WPR), jnp.int32),
            pltpu.VMEM((_BPW // 2, EMB_DIM), jnp.float32),
            pltpu.SemaphoreType.DMA,
        ],
    )
    def tower(idx_hbm, tab_w, outb, idx_v, rows_v, out_v, sem):
        wid = lax.axis_index("s") * _NC + lax.axis_index("c")
        base = wid * _BPW
        half = _BPW // 2
        lanes = lax.iota(jnp.int32, _L)

        pltpu.sync_copy(idx_hbm.at[pl.ds(base, _BPW)], idx_v)

        for ch in range(2):
            off = ch * half

            def issue(g, _, off=off):
                iv = idx_v[pl.ds(off + g * _L, _L)]
                for t in range(_L):
                    pltpu.async_copy(
                        tab_w.at[iv[t]], rows_v.at[g * _L + t], sem)
                return _

            lax.fori_loop(0, half // _L, issue, None)
            pltpu.make_async_copy(
                tab_w.at[pl.ds(0, half)], rows_v, sem).wait()

            def row_body(rr, _):
                for u in range(4):
                    r = rr * 4 + u
                    w0 = rows_v[r, pl.ds(0, _L)]
                    w1 = rows_v[r, pl.ds(_L, _L)]
                    a0, b0 = _unpack_pair(w0)
                    a1, b1 = _unpack_pair(w1)
                    acc = a0 * a0 + b0 * b0 + a1 * a1 + b1 * b1
                    for k in (1, 2, 4, 8):
                        acc = acc + _shuffle_xor(acc, lanes, k)
                    inv = _rsqrt16(acc)
                    out_v[r, pl.ds(0 * _L, _L)] = a0 * inv
                    out_v[r, pl.ds(1 * _L, _L)] = a1 * inv
                    out_v[r, pl.ds(2 * _L, _L)] = b0 * inv
                    out_v[r, pl.ds(3 * _L, _L)] = b1 * inv
                return _

            lax.fori_loop(0, half // 4, row_body, None)
            pltpu.sync_copy(out_v, outb.at[pl.ds(base + off, half)])

    return tower


def _pack_cols(table):
    n, d = table.shape
    e = lax.slice(table, (0, 0), (n, d), (1, 2))
    o = lax.slice(table, (0, 1), (n, d), (1, 2))
    be = lax.bitcast_convert_type(e, jnp.int32) + jnp.int32(0x8000)
    bo = lax.bitcast_convert_type(o, jnp.int32) + jnp.int32(0x8000)
    return lax.bitwise_or(lax.shift_right_logical(be, 16),
                          lax.bitwise_and(bo, jnp.int32(_MASKHI)))


_UNSCRAMBLE = np.arange(EMB_DIM) // 2 + _WPR * (np.arange(EMB_DIM) % 2)


def kernel(user_idx, item_idx, user_table, item_table):
    tower = _make_tower_kernel()
    u = tower(user_idx, _pack_cols(user_table))
    v = tower(item_idx, _pack_cols(item_table))
    out = jnp.stack([u, v], axis=0)
    return jnp.take(out, jnp.asarray(_UNSCRAMBLE), axis=-1)


# speedup vs baseline: 5.2938x; 5.2938x over previous
import functools

import jax
import jax.numpy as jnp
from jax import lax
from jax.experimental import pallas as pl
from jax.experimental.pallas import tpu as pltpu
from jax.experimental.pallas import tpu_sc as plsc

NUM_USERS = 1000000
NUM_ITEMS = 1000000
EMB_DIM = 64
BATCH = 16384

_NC = 2
_NS = 16
_L = 16
_NW = _NC * _NS
_BPW = BATCH // _NW
_NSEM = 8


def _rsqrt16(s):
    s = jnp.maximum(s, jnp.float32(1e-24))
    i = lax.bitcast_convert_type(s, jnp.int32)
    i = jnp.int32(0x5F3759DF) - lax.shift_right_logical(i, 1)
    y = lax.bitcast_convert_type(i, jnp.float32)
    for _ in range(2):
        y = y * (jnp.float32(1.5) - jnp.float32(0.5) * s * y * y)
    return y


def _shuffle_xor(x, lanes, k):
    idx = lax.bitwise_xor(lanes, jnp.int32(k))
    return lax.gather(
        x, idx[:, None],
        dimension_numbers=lax.GatherDimensionNumbers(
            offset_dims=(), collapsed_slice_dims=(0,), start_index_map=(0,)),
        slice_sizes=(1,),
        mode=lax.GatherScatterMode.PROMISE_IN_BOUNDS)


@functools.cache
def _make_sc_kernel():
    mesh = plsc.VectorSubcoreMesh(core_axis_name="c", subcore_axis_name="s")
    _QS = EMB_DIM // _L

    @functools.partial(
        pl.kernel,
        mesh=mesh,
        out_type=jax.ShapeDtypeStruct((2, BATCH, EMB_DIM), jnp.float32),
        scratch_types=[
            pltpu.VMEM((_BPW,), jnp.int32),
            pltpu.VMEM((_BPW,), jnp.int32),
            pltpu.VMEM((_BPW, EMB_DIM), jnp.float32),
        ] + [pltpu.SemaphoreType.DMA] * _NSEM,
    )
    def two_tower(user_idx, item_idx, user_table, item_table, out,
                  uidx_v, iidx_v, rows_v, *sems):
        wid = lax.axis_index("s") * _NC + lax.axis_index("c")
        base = wid * _BPW
        lanes = lax.iota(jnp.int32, _L)

        pltpu.sync_copy(user_idx.at[pl.ds(base, _BPW)], uidx_v)
        pltpu.sync_copy(item_idx.at[pl.ds(base, _BPW)], iidx_v)

        def normalize_rows():
            def row_body(rr, _):
                for u in range(4):
                    r = rr * 4 + u
                    vs = [rows_v[r, pl.ds(q * _L, _L)] for q in range(_QS)]
                    acc = vs[0] * vs[0]
                    for q in range(1, _QS):
                        acc = acc + vs[q] * vs[q]
                    for k in (1, 2, 4, 8):
                        acc = acc + _shuffle_xor(acc, lanes, k)
                    inv = _rsqrt16(acc)
                    for q in range(_QS):
                        rows_v[r, pl.ds(q * _L, _L)] = vs[q] * inv
                return _
            lax.fori_loop(0, _BPW // 4, row_body, None)

        for tower, tab, idx_v in ((0, user_table, uidx_v),
                                  (1, item_table, iidx_v)):
            def issue(g, _, tab=tab, idx_v=idx_v):
                iv = idx_v[pl.ds(g * _L, _L)]
                for k in range(_L):
                    pltpu.async_copy(
                        tab.at[iv[k]], rows_v.at[g * _L + k],
                        sems[k % _NSEM])
                return _

            lax.fori_loop(0, _BPW // _L, issue, None)
            for s in range(_NSEM):
                pltpu.make_async_copy(
                    tab.at[pl.ds(0, _BPW // _NSEM)],
                    rows_v.at[pl.ds(0, _BPW // _NSEM)], sems[s]).wait()
            normalize_rows()
            pltpu.sync_copy(rows_v, out.at[tower, pl.ds(base, _BPW)])

    return two_tower


def kernel(user_idx, item_idx, user_table, item_table):
    return _make_sc_kernel()(user_idx, item_idx, user_table, item_table)
